# baseline (speedup 1.0000x reference)
import dataclasses
import functools

import jax
import jax.numpy as jnp
from jax import lax
from jax.experimental import pallas as pl
from jax.experimental.pallas import tpu as pltpu
from jax.experimental.pallas import tpu_sc as plsc

N = 10000
NP = 10112
E = 320000
NC = 2
NS = 16
NW = NC * NS
CH = 80
CPT = 131
E_PAD = NW * CH * CPT
PACK_SHIFT = 14
ROWS_PER_TEC = NP // NS
NEG = -1e30
D1 = 64
D1E = 72
D2 = 40
D2E = 48


def _f32(shape):
    return jax.ShapeDtypeStruct(shape, jnp.float32)


def _fill_row(h_ref, h, asrc_col, d, de):
    h_ref[:N, :d] = h
    h_ref[N:, :d] = jnp.zeros((NP - N, d), jnp.float32)
    h_ref[:, d:d + 1] = jnp.ones((NP, 1), jnp.float32)
    h_ref[:N, d + 1:d + 2] = asrc_col
    h_ref[N:, d + 1:d + 2] = jnp.full((NP - N, 1), NEG, jnp.float32)
    h_ref[:, d + 2:] = jnp.zeros((NP, de - d - 2), jnp.float32)


def _tc1_body(x_ref, w_ref, as_ref, ad_ref, h_ref, adst_ref, m_ref):
    h = jnp.dot(x_ref[...], w_ref[...], preferred_element_type=jnp.float32)
    asrc_col = jnp.dot(h, as_ref[...][:, None],
                       preferred_element_type=jnp.float32)
    _fill_row(h_ref, h, asrc_col, D1, D1E)
    adst = jnp.sum(h * ad_ref[...], axis=1)
    adst_ref[0:1, :N] = adst[None, :]
    adst_ref[0:1, N:] = jnp.full((1, NP - N), NEG, jnp.float32)
    mm = jnp.max(asrc_col) + jnp.max(adst)
    m = jnp.maximum(mm, 0.2 * mm)
    m_ref[0:1, :] = jnp.full((1, 16), m, jnp.float32)


def _tc1(x, W1, a_src1, a_dst1):
    return pl.pallas_call(
        _tc1_body,
        out_shape=(_f32((NP, D1E)), _f32((1, NP)), _f32((1, 16))),
    )(x, W1, a_src1, a_dst1)


def _tc2_body(acc_ref, b_ref, w_ref, as_ref, ad_ref,
              h_ref, adst_ref, m_ref):
    acc = acc_ref[0] + acc_ref[1]
    h1 = acc[:N, :D1] / acc[:N, D1:D1 + 1] + b_ref[...]
    h1 = jnp.maximum(h1, 0.0)
    h2 = jnp.dot(h1, w_ref[...], preferred_element_type=jnp.float32)
    asrc_col = jnp.dot(h2, as_ref[...][:, None],
                       preferred_element_type=jnp.float32)
    _fill_row(h_ref, h2, asrc_col, D2, D2E)
    adst = jnp.sum(h2 * ad_ref[...], axis=1)
    adst_ref[0:1, :N] = adst[None, :]
    adst_ref[0:1, N:] = jnp.full((1, NP - N), NEG, jnp.float32)
    mm = jnp.max(asrc_col) + jnp.max(adst)
    m = jnp.maximum(mm, 0.2 * mm)
    m_ref[0:1, :] = jnp.full((1, 16), m, jnp.float32)


def _tc2(acc, b1, W2, a_src2, a_dst2):
    return pl.pallas_call(
        _tc2_body,
        out_shape=(_f32((NP, D2E)), _f32((1, NP)), _f32((1, 16))),
    )(acc, b1, W2, a_src2, a_dst2)


def _tc3_body(acc_ref, b_ref, out_ref):
    acc = acc_ref[0] + acc_ref[1]
    v = acc[:N, :D2] / acc[:N, D2:D2 + 1] + b_ref[...]
    v = v - jnp.max(v, axis=1, keepdims=True)
    out_ref[...] = v - jnp.log(jnp.sum(jnp.exp(v), axis=1, keepdims=True))


def _tc3(acc, b2):
    return pl.pallas_call(
        _tc3_body,
        out_shape=_f32((N, D2)),
    )(acc, b2)


def _sc_compiler_params():
    cp = pltpu.CompilerParams()
    fields = pltpu.CompilerParams.__dataclass_fields__
    if "needs_layout_passes" in fields:
        cp = dataclasses.replace(cp, needs_layout_passes=False)
    if "use_tc_tiling_on_sc" in fields:
        cp = dataclasses.replace(cp, use_tc_tiling_on_sc=False)
    return cp


def _sc_edges(h_pad, adst, m16, edges_packed, zer, d, de, h_spmem=True):
    mesh = plsc.VectorSubcoreMesh(core_axis_name="c", subcore_axis_name="s")

    @functools.partial(
        pl.kernel,
        out_type=_f32((NC, NP, de)),
        mesh=mesh,
        compiler_params=_sc_compiler_params(),
        scratch_types=[
            pltpu.VMEM_SHARED((NP, de), jnp.float32),
            pltpu.VMEM_SHARED((NP, de), jnp.float32),
            pltpu.VMEM((CPT + 2, CH), jnp.int32),
            pltpu.VMEM((CH,), jnp.int32),
            pltpu.VMEM((CH,), jnp.int32),
            pltpu.VMEM((CH,), jnp.int32),
            pltpu.VMEM((CH,), jnp.int32),
            pltpu.VMEM((CH,), jnp.int32),
            pltpu.VMEM((CH,), jnp.int32),
            pltpu.VMEM((CH, de), jnp.float32),
            pltpu.VMEM((CH, de), jnp.float32),
            pltpu.VMEM((CH, de), jnp.float32),
            pltpu.VMEM((NP,), jnp.float32),
            pltpu.VMEM((16,), jnp.float32),
            pltpu.SemaphoreType.DMA,
            pltpu.SemaphoreType.DMA,
            pltpu.SemaphoreType.DMA,
            pltpu.SemaphoreType.DMA,
            pltpu.SemaphoreType.DMA,
            pltpu.SemaphoreType.DMA,
        ],
    )
    def k(h_hbm, adst_hbm, m_hbm, ep_hbm, zer_hbm, acc_hbm,
          acc_sh, h_sh, ep_all, srcb0, srcb1, srcb2, dstb0, dstb1, dstb2,
          rows0, rows1, rows2,
          adst_v, m_v, gsem0, gsem1, gsem2, ssem0, ssem1, ssem2):
        cid = lax.axis_index("c")
        sid = lax.axis_index("s")
        wid = cid * NS + sid
        pltpu.sync_copy(adst_hbm, adst_v)
        pltpu.sync_copy(m_hbm, m_v)
        pltpu.sync_copy(ep_hbm.at[wid], ep_all)
        row0 = sid * ROWS_PER_TEC
        if h_spmem:
            pltpu.sync_copy(h_hbm.at[pl.ds(row0, ROWS_PER_TEC)],
                            h_sh.at[pl.ds(row0, ROWS_PER_TEC)])
        h_tab = h_sh if h_spmem else h_hbm
        pltpu.sync_copy(zer_hbm.at[pl.ds(row0, ROWS_PER_TEC)],
                        acc_sh.at[pl.ds(row0, ROWS_PER_TEC)])
        m_reg = m_v[...]
        mask14 = jnp.full((16,), (1 << PACK_SHIFT) - 1, jnp.int32)
        rows = (rows0, rows1, rows2)
        srcb = (srcb0, srcb1, srcb2)
        dstb = (dstb0, dstb1, dstb2)
        gsem = (gsem0, gsem1, gsem2)
        ssem = (ssem0, ssem1, ssem2)

        def unpack_src(ci, b):
            @pl.loop(0, CH, step=16)
            def _(g):
                p = ep_all[ci, pl.ds(g, 16)]
                srcb[b][pl.ds(g, 16)] = jnp.bitwise_and(p, mask14)

        def issue_gather(ci, b):
            unpack_src(ci, b)
            pltpu.async_copy(h_tab.at[srcb[b]], rows[b], gsem[b])

        def compute(ci, b):
            pltpu.make_async_copy(h_tab.at[srcb[b]], rows[b],
                                  gsem[b]).wait()
            rows_v = rows[b]

            @pl.loop(0, CH, step=16)
            def _(g):
                p = ep_all[ci, pl.ds(g, 16)]
                dst16 = lax.shift_right_logical(p, PACK_SHIFT)
                dstb[b][pl.ds(g, 16)] = dst16
                row16 = lax.iota(jnp.int32, 16) + g
                s = plsc.load_gather(
                    rows_v, [row16, jnp.full((16,), d + 1, jnp.int32)])
                dd = plsc.load_gather(adst_v, [dst16])
                e = s + dd
                e = jnp.maximum(e, 0.2 * e)
                w16 = jnp.exp(e - m_reg)
                for c in range(d + 1):
                    col16 = jnp.full((16,), c, jnp.int32)
                    v = plsc.load_gather(rows_v, [row16, col16])
                    plsc.store_scatter(rows_v, [row16, col16], v * w16)

            pltpu.async_copy(rows_v, acc_sh.at[dstb[b]], ssem[b],
                             add=True)

        def wait_scatter(b):
            pltpu.make_async_copy(rows[b], acc_sh.at[dstb[b]],
                                  ssem[b]).wait()

        plsc.subcore_barrier()

        issue_gather(0, 0)
        issue_gather(1, 1)
        compute(0, 0)
        issue_gather(2, 2)
        compute(1, 1)
        wait_scatter(0)
        issue_gather(3, 0)

        @pl.loop(2, CPT, step=3)
        def _(ci0):
            for k in range(3):
                ci = ci0 + k
                b = (2 + k) % 3
                compute(ci, b)
                wait_scatter((b + 2) % 3)
                issue_gather(ci + 2, (b + 2) % 3)

        wait_scatter((CPT - 1) % 3)
        pltpu.make_async_copy(h_tab.at[srcb[CPT % 3]], rows[CPT % 3],
                              gsem[CPT % 3]).wait()
        pltpu.make_async_copy(h_tab.at[srcb[(CPT + 1) % 3]],
                              rows[(CPT + 1) % 3],
                              gsem[(CPT + 1) % 3]).wait()
        plsc.subcore_barrier()
        pltpu.sync_copy(acc_sh.at[pl.ds(row0, ROWS_PER_TEC)],
                        acc_hbm.at[cid].at[pl.ds(row0, ROWS_PER_TEC)])

    return k(h_pad, adst, m16, edges_packed, zer)


def kernel(x, edges_index, W1, a_src1, a_dst1, b1, W2, a_src2, a_dst2, b2):
    loop = jnp.arange(N, dtype=edges_index.dtype)
    pad = jnp.full((E_PAD - E - N,), N, dtype=edges_index.dtype)
    src = jnp.concatenate([edges_index[0], loop, pad])
    dst = jnp.concatenate([edges_index[1], loop, pad])
    sent = N | (N << PACK_SHIFT)
    packed = (src | (dst << PACK_SHIFT)).reshape(NW, CPT, CH)
    tail = jnp.full((NW, 2, CH), sent, dtype=packed.dtype)
    packed = jnp.concatenate([packed, tail], axis=1)

    zer1 = jnp.zeros((NP, D1E), jnp.float32)
    zer2 = jnp.zeros((NP, D2E), jnp.float32)

    h1, adst1, m1 = _tc1(x, W1, a_src1, a_dst1)
    acc1 = _sc_edges(h1, adst1.reshape(NP), m1.reshape(16),
                     packed, zer1, D1, D1E, h_spmem=False)
    h2, adst2, m2 = _tc2(acc1, b1, W2, a_src2, a_dst2)
    acc2 = _sc_edges(h2, adst2.reshape(NP), m2.reshape(16),
                     packed, zer2, D2, D2E)
    return _tc3(acc2, b2)


# speedup vs baseline: 1.2345x; 1.2345x over previous
import dataclasses
import functools

import jax
import jax.numpy as jnp
from jax import lax
from jax.experimental import pallas as pl
from jax.experimental.pallas import tpu as pltpu
from jax.experimental.pallas import tpu_sc as plsc

N = 10000
NP = 10112
E = 320000
NC = 2
NS = 16
NW = NC * NS
CH = 80
CPT = 131
E_PAD = NW * CH * CPT
PACK_SHIFT = 14
ROWS_PER_TEC = NP // NS
NEG = -1e30
D1 = 64
D1E = 72
D2 = 40
D2E = 48


def _f32(shape):
    return jax.ShapeDtypeStruct(shape, jnp.float32)


def _fill_row(h_ref, h, asrc_col, d, de):
    h_ref[:N, :d] = h
    h_ref[N:, :d] = jnp.zeros((NP - N, d), jnp.float32)
    h_ref[:, d:d + 1] = jnp.ones((NP, 1), jnp.float32)
    h_ref[:N, d + 1:d + 2] = asrc_col
    h_ref[N:, d + 1:d + 2] = jnp.full((NP - N, 1), NEG, jnp.float32)
    h_ref[:, d + 2:] = jnp.zeros((NP, de - d - 2), jnp.float32)


def _tc1_body(x_ref, w_ref, as_ref, ad_ref, h_ref, adst_ref, m_ref):
    h = jnp.dot(x_ref[...], w_ref[...], preferred_element_type=jnp.float32)
    asrc_col = jnp.dot(h, as_ref[...][:, None],
                       preferred_element_type=jnp.float32)
    _fill_row(h_ref, h, asrc_col, D1, D1E)
    adst = jnp.sum(h * ad_ref[...], axis=1)
    adst_ref[0:1, :N] = adst[None, :]
    adst_ref[0:1, N:] = jnp.full((1, NP - N), NEG, jnp.float32)
    mm = jnp.max(asrc_col) + jnp.max(adst)
    m = jnp.maximum(mm, 0.2 * mm)
    m_ref[0:1, :] = jnp.full((1, 16), m, jnp.float32)


def _tc1(x, W1, a_src1, a_dst1):
    return pl.pallas_call(
        _tc1_body,
        out_shape=(_f32((NP, D1E)), _f32((1, NP)), _f32((1, 16))),
    )(x, W1, a_src1, a_dst1)


def _tc2_body(acc_ref, b_ref, w_ref, as_ref, ad_ref,
              h_ref, adst_ref, m_ref):
    acc = acc_ref[0] + acc_ref[1]
    h1 = acc[:N, :D1] / acc[:N, D1:D1 + 1] + b_ref[...]
    h1 = jnp.maximum(h1, 0.0)
    h2 = jnp.dot(h1, w_ref[...], preferred_element_type=jnp.float32)
    asrc_col = jnp.dot(h2, as_ref[...][:, None],
                       preferred_element_type=jnp.float32)
    _fill_row(h_ref, h2, asrc_col, D2, D2E)
    adst = jnp.sum(h2 * ad_ref[...], axis=1)
    adst_ref[0:1, :N] = adst[None, :]
    adst_ref[0:1, N:] = jnp.full((1, NP - N), NEG, jnp.float32)
    mm = jnp.max(asrc_col) + jnp.max(adst)
    m = jnp.maximum(mm, 0.2 * mm)
    m_ref[0:1, :] = jnp.full((1, 16), m, jnp.float32)


def _tc2(acc, b1, W2, a_src2, a_dst2):
    return pl.pallas_call(
        _tc2_body,
        out_shape=(_f32((NP, D2E)), _f32((1, NP)), _f32((1, 16))),
    )(acc, b1, W2, a_src2, a_dst2)


def _tc3_body(acc_ref, b_ref, out_ref):
    acc = acc_ref[0] + acc_ref[1]
    v = acc[:N, :D2] / acc[:N, D2:D2 + 1] + b_ref[...]
    v = v - jnp.max(v, axis=1, keepdims=True)
    out_ref[...] = v - jnp.log(jnp.sum(jnp.exp(v), axis=1, keepdims=True))


def _tc3(acc, b2):
    return pl.pallas_call(
        _tc3_body,
        out_shape=_f32((N, D2)),
    )(acc, b2)


def _sc_compiler_params():
    cp = pltpu.CompilerParams()
    fields = pltpu.CompilerParams.__dataclass_fields__
    if "needs_layout_passes" in fields:
        cp = dataclasses.replace(cp, needs_layout_passes=False)
    if "use_tc_tiling_on_sc" in fields:
        cp = dataclasses.replace(cp, use_tc_tiling_on_sc=False)
    return cp


def _sc_edges(h_pad, adst, m16, edges_packed, zer, d, de, h_spmem=True):
    mesh = plsc.VectorSubcoreMesh(core_axis_name="c", subcore_axis_name="s")

    @functools.partial(
        pl.kernel,
        out_type=_f32((NC, NP, de)),
        mesh=mesh,
        compiler_params=_sc_compiler_params(),
        scratch_types=[
            pltpu.VMEM_SHARED((NP, de), jnp.float32),
            pltpu.VMEM_SHARED((NP, de), jnp.float32),
            pltpu.VMEM((CPT + 2, CH), jnp.int32),
            pltpu.VMEM((CH,), jnp.int32),
            pltpu.VMEM((CH,), jnp.int32),
            pltpu.VMEM((CH,), jnp.int32),
            pltpu.VMEM((CH,), jnp.int32),
            pltpu.VMEM((CH,), jnp.int32),
            pltpu.VMEM((CH,), jnp.int32),
            pltpu.VMEM((CH, de), jnp.float32),
            pltpu.VMEM((CH, de), jnp.float32),
            pltpu.VMEM((CH, de), jnp.float32),
            pltpu.VMEM((NP,), jnp.float32),
            pltpu.VMEM((16,), jnp.float32),
            pltpu.SemaphoreType.DMA,
            pltpu.SemaphoreType.DMA,
            pltpu.SemaphoreType.DMA,
            pltpu.SemaphoreType.DMA,
            pltpu.SemaphoreType.DMA,
            pltpu.SemaphoreType.DMA,
        ],
    )
    def k(h_hbm, adst_hbm, m_hbm, ep_hbm, zer_hbm, acc_hbm,
          acc_sh, h_sh, ep_all, srcb0, srcb1, srcb2, dstb0, dstb1, dstb2,
          rows0, rows1, rows2,
          adst_v, m_v, gsem0, gsem1, gsem2, ssem0, ssem1, ssem2):
        cid = lax.axis_index("c")
        sid = lax.axis_index("s")
        wid = cid * NS + sid
        row0 = sid * ROWS_PER_TEC
        d0 = pltpu.async_copy(adst_hbm, adst_v, gsem0)
        d1 = pltpu.async_copy(m_hbm, m_v, gsem1)
        d2 = pltpu.async_copy(ep_hbm.at[wid], ep_all, gsem2)
        d3 = pltpu.async_copy(zer_hbm.at[pl.ds(row0, ROWS_PER_TEC)],
                              acc_sh.at[pl.ds(row0, ROWS_PER_TEC)], ssem0)
        if h_spmem:
            d4 = pltpu.async_copy(h_hbm.at[pl.ds(row0, ROWS_PER_TEC)],
                                  h_sh.at[pl.ds(row0, ROWS_PER_TEC)], ssem1)
            d4.wait()
        h_tab = h_sh if h_spmem else h_hbm
        d0.wait()
        d1.wait()
        d2.wait()
        d3.wait()
        m_reg = m_v[...]
        mask14 = jnp.full((16,), (1 << PACK_SHIFT) - 1, jnp.int32)
        rows = (rows0, rows1, rows2)
        srcb = (srcb0, srcb1, srcb2)
        dstb = (dstb0, dstb1, dstb2)
        gsem = (gsem0, gsem1, gsem2)
        ssem = (ssem0, ssem1, ssem2)

        def unpack_src(ci, b):
            @pl.loop(0, CH, step=16)
            def _(g):
                p = ep_all[ci, pl.ds(g, 16)]
                srcb[b][pl.ds(g, 16)] = jnp.bitwise_and(p, mask14)

        def issue_gather(ci, b):
            unpack_src(ci, b)
            pltpu.async_copy(h_tab.at[srcb[b]], rows[b], gsem[b])

        def compute(ci, b):
            pltpu.make_async_copy(h_tab.at[srcb[b]], rows[b],
                                  gsem[b]).wait()
            rows_v = rows[b]

            @pl.loop(0, CH, step=16)
            def _(g):
                p = ep_all[ci, pl.ds(g, 16)]
                dst16 = lax.shift_right_logical(p, PACK_SHIFT)
                dstb[b][pl.ds(g, 16)] = dst16
                row16 = lax.iota(jnp.int32, 16) + g
                s = plsc.load_gather(
                    rows_v, [row16, jnp.full((16,), d + 1, jnp.int32)])
                dd = plsc.load_gather(adst_v, [dst16])
                e = s + dd
                e = jnp.maximum(e, 0.2 * e)
                w16 = jnp.exp(e - m_reg)
                for c in range(d + 1):
                    col16 = jnp.full((16,), c, jnp.int32)
                    v = plsc.load_gather(rows_v, [row16, col16])
                    plsc.store_scatter(rows_v, [row16, col16], v * w16)

            pltpu.async_copy(rows_v, acc_sh.at[dstb[b]], ssem[b],
                             add=True)

        def wait_scatter(b):
            pltpu.make_async_copy(rows[b], acc_sh.at[dstb[b]],
                                  ssem[b]).wait()

        plsc.subcore_barrier()

        issue_gather(0, 0)
        issue_gather(1, 1)
        compute(0, 0)
        issue_gather(2, 2)
        compute(1, 1)
        wait_scatter(0)
        issue_gather(3, 0)

        @pl.loop(2, CPT, step=3)
        def _(ci0):
            for k in range(3):
                ci = ci0 + k
                b = (2 + k) % 3
                compute(ci, b)
                wait_scatter((b + 2) % 3)
                issue_gather(ci + 2, (b + 2) % 3)

        wait_scatter((CPT - 1) % 3)
        pltpu.make_async_copy(h_tab.at[srcb[CPT % 3]], rows[CPT % 3],
                              gsem[CPT % 3]).wait()
        pltpu.make_async_copy(h_tab.at[srcb[(CPT + 1) % 3]],
                              rows[(CPT + 1) % 3],
                              gsem[(CPT + 1) % 3]).wait()
        plsc.subcore_barrier()
        pltpu.sync_copy(acc_sh.at[pl.ds(row0, ROWS_PER_TEC)],
                        acc_hbm.at[cid].at[pl.ds(row0, ROWS_PER_TEC)])

    return k(h_pad, adst, m16, edges_packed, zer)


def kernel(x, edges_index, W1, a_src1, a_dst1, b1, W2, a_src2, a_dst2, b2):
    loop = jnp.arange(N, dtype=edges_index.dtype)
    pad = jnp.full((E_PAD - E - N,), N, dtype=edges_index.dtype)
    src = jnp.concatenate([edges_index[0], loop, pad])
    dst = jnp.concatenate([edges_index[1], loop, pad])
    sent = N | (N << PACK_SHIFT)
    packed = (src | (dst << PACK_SHIFT)).reshape(NW, CPT, CH)
    tail = jnp.full((NW, 2, CH), sent, dtype=packed.dtype)
    packed = jnp.concatenate([packed, tail], axis=1)

    zer1 = jnp.zeros((NP, D1E), jnp.float32)
    zer2 = jnp.zeros((NP, D2E), jnp.float32)

    h1, adst1, m1 = _tc1(x, W1, a_src1, a_dst1)
    acc1 = _sc_edges(h1, adst1.reshape(NP), m1.reshape(16),
                     packed, zer1, D1, D1E)
    h2, adst2, m2 = _tc2(acc1, b1, W2, a_src2, a_dst2)
    acc2 = _sc_edges(h2, adst2.reshape(NP), m2.reshape(16),
                     packed, zer2, D2, D2E)
    return _tc3(acc2, b2)
